# baseline (speedup 1.0000x reference)
import functools

import jax
import jax.numpy as jnp
from jax import lax
from jax.experimental import pallas as pl
from jax.experimental.pallas import tpu as pltpu
from jax.experimental.pallas import tpu_sc as plsc

NUM_EMB = 1000000
EMB_DIM = 32
BATCH = 16384
HIST = 50
B = BATCH * HIST

NC, NS = 2, 16
NW = NC * NS
BATCH_PW = BATCH // NW
NSTREAM = 4
NSTEP = BATCH_PW // NSTREAM

MLP_ROWS = 8000


def _sc_gather(table, x):
    mesh = plsc.VectorSubcoreMesh(core_axis_name="c", subcore_axis_name="s")

    @functools.partial(
        pl.kernel,
        out_type=jax.ShapeDtypeStruct((BATCH, HIST, EMB_DIM), jnp.float32),
        mesh=mesh,
        compiler_params=pltpu.CompilerParams(use_tc_tiling_on_sc=False),
        scratch_types=[
            pltpu.VMEM((BATCH_PW, HIST), jnp.int32),
            pltpu.VMEM((2, NSTREAM, HIST, EMB_DIM), jnp.float32),
            pltpu.SemaphoreType.DMA,
            pltpu.SemaphoreType.DMA,
        ],
    )
    def k(table_hbm, x_hbm, out_hbm, idx_v, rows_v, gsem, psem):
        wid = lax.axis_index("s") * NC + lax.axis_index("c")
        bbase = wid * BATCH_PW
        pltpu.sync_copy(x_hbm.at[pl.ds(bbase, BATCH_PW)], idx_v)

        def step(s, _):
            for t in range(NSTREAM):
                pltpu.async_copy(
                    table_hbm.at[idx_v.at[s * NSTREAM + t]],
                    rows_v.at[0].at[t],
                    gsem,
                )
            for t in range(NSTREAM):
                pltpu.make_async_copy(
                    table_hbm.at[idx_v.at[0]],
                    rows_v.at[0].at[0],
                    gsem,
                ).wait()
            pltpu.async_copy(
                rows_v.at[0],
                out_hbm.at[pl.ds(bbase + s * NSTREAM, NSTREAM)],
                psem,
            ).wait()
            return 0

        lax.fori_loop(0, NSTEP, step, 0)

    return k(table, x)


def _mlp_body(x_ref, w1_ref, b1_ref, w2_ref, b2_ref, o_ref):
    h = jnp.dot(x_ref[...], w1_ref[...], preferred_element_type=jnp.float32)
    h = jnp.maximum(h + b1_ref[...], 0.0)
    h = jnp.dot(h, w2_ref[...], preferred_element_type=jnp.float32)
    o_ref[...] = jnp.maximum(h + b2_ref[...], 0.0)


def _tc_mlp(emb, w1t, b1r, w2t, b2r):
    m = emb.shape[0]
    grid = (m // MLP_ROWS,)
    return pl.pallas_call(
        _mlp_body,
        grid=grid,
        in_specs=[
            pl.BlockSpec((MLP_ROWS, EMB_DIM), lambda i: (i, 0)),
            pl.BlockSpec((EMB_DIM, EMB_DIM), lambda i: (0, 0)),
            pl.BlockSpec((1, EMB_DIM), lambda i: (0, 0)),
            pl.BlockSpec((EMB_DIM, EMB_DIM), lambda i: (0, 0)),
            pl.BlockSpec((1, EMB_DIM), lambda i: (0, 0)),
        ],
        out_specs=pl.BlockSpec((MLP_ROWS, EMB_DIM), lambda i: (i, 0)),
        out_shape=jax.ShapeDtypeStruct((m, EMB_DIM), jnp.float32),
    )(emb, w1t, b1r, w2t, b2r)


def kernel(x, emb, W1, b1, W2, b2):
    t = _tc_mlp(emb, W1.T, b1.reshape(1, EMB_DIM), W2.T, b2.reshape(1, EMB_DIM))
    return _sc_gather(t, x.astype(jnp.int32))


# speedup vs baseline: 1.0040x; 1.0040x over previous
import functools

import jax
import jax.numpy as jnp
from jax import lax
from jax.experimental import pallas as pl
from jax.experimental.pallas import tpu as pltpu
from jax.experimental.pallas import tpu_sc as plsc

NUM_EMB = 1000000
EMB_DIM = 32
BATCH = 16384
HIST = 50
B = BATCH * HIST

NC, NS = 2, 16
NW = NC * NS
BATCH_PW = BATCH // NW
NSTREAM = 8
NSTEP = BATCH_PW // NSTREAM
CHUNK = NSTREAM * HIST

MLP_BATCH = 64
MLP_ROWS = MLP_BATCH * HIST


def _sc_gather(table, x):
    mesh = plsc.VectorSubcoreMesh(core_axis_name="c", subcore_axis_name="s")

    @functools.partial(
        pl.kernel,
        out_type=jax.ShapeDtypeStruct((B, EMB_DIM), jnp.float32),
        mesh=mesh,
        compiler_params=pltpu.CompilerParams(use_tc_tiling_on_sc=False),
        scratch_types=[
            pltpu.VMEM((BATCH_PW, HIST), jnp.int32),
            pltpu.VMEM((2, CHUNK, EMB_DIM), jnp.float32),
            pltpu.SemaphoreType.DMA,
            pltpu.SemaphoreType.DMA,
        ],
    )
    def k(table_hbm, x_hbm, out_hbm, idx_v, rows_v, gsem, psem):
        wid = lax.axis_index("s") * NC + lax.axis_index("c")
        rowbase = wid * BATCH_PW * HIST
        pltpu.sync_copy(x_hbm.at[pl.ds(wid * BATCH_PW, BATCH_PW)], idx_v)

        def step(s, _):
            for t in range(NSTREAM):
                pltpu.async_copy(
                    table_hbm.at[idx_v.at[s * NSTREAM + t]],
                    rows_v.at[0].at[pl.ds(t * HIST, HIST)],
                    gsem,
                )
            for t in range(NSTREAM):
                pltpu.make_async_copy(
                    table_hbm.at[idx_v.at[0]],
                    rows_v.at[0].at[pl.ds(0, HIST)],
                    gsem,
                ).wait()
            pltpu.async_copy(
                rows_v.at[0],
                out_hbm.at[pl.ds(rowbase + s * CHUNK, CHUNK)],
                psem,
            ).wait()
            return 0

        lax.fori_loop(0, NSTEP, step, 0)

    return k(table, x)


def _mlp_body(x_ref, w1_ref, b1_ref, w2_ref, b2_ref, o_ref):
    h = jnp.dot(x_ref[...], w1_ref[...], preferred_element_type=jnp.float32)
    h = jnp.maximum(h + b1_ref[...], 0.0)
    h = jnp.dot(h, w2_ref[...], preferred_element_type=jnp.float32)
    h = jnp.maximum(h + b2_ref[...], 0.0)
    o_ref[...] = h.reshape(MLP_BATCH, HIST, EMB_DIM)


def _tc_mlp(g, w1t, b1r, w2t, b2r):
    grid = (BATCH // MLP_BATCH,)
    return pl.pallas_call(
        _mlp_body,
        grid=grid,
        in_specs=[
            pl.BlockSpec((MLP_ROWS, EMB_DIM), lambda i: (i, 0)),
            pl.BlockSpec((EMB_DIM, EMB_DIM), lambda i: (0, 0)),
            pl.BlockSpec((1, EMB_DIM), lambda i: (0, 0)),
            pl.BlockSpec((EMB_DIM, EMB_DIM), lambda i: (0, 0)),
            pl.BlockSpec((1, EMB_DIM), lambda i: (0, 0)),
        ],
        out_specs=pl.BlockSpec((MLP_BATCH, HIST, EMB_DIM), lambda i: (i, 0, 0)),
        out_shape=jax.ShapeDtypeStruct((BATCH, HIST, EMB_DIM), jnp.float32),
    )(g, w1t, b1r, w2t, b2r)


def kernel(x, emb, W1, b1, W2, b2):
    g = _sc_gather(emb, x.astype(jnp.int32))
    return _tc_mlp(g, W1.T, b1.reshape(1, EMB_DIM), W2.T, b2.reshape(1, EMB_DIM))


# speedup vs baseline: 1.2500x; 1.2450x over previous
import functools

import jax
import jax.numpy as jnp
from jax import lax
from jax.experimental import pallas as pl
from jax.experimental.pallas import tpu as pltpu
from jax.experimental.pallas import tpu_sc as plsc

NUM_EMB = 1000000
EMB_DIM = 32
BATCH = 16384
HIST = 50
B = BATCH * HIST

NC, NS = 2, 16
NW = NC * NS
BATCH_PW = BATCH // NW
IDXW = 128
NSTREAM = BATCH_PW // IDXW

MLP_B = 2048


def _sc_gather(table, xt):
    mesh = plsc.VectorSubcoreMesh(core_axis_name="c", subcore_axis_name="s")

    @functools.partial(
        pl.kernel,
        out_type=jax.ShapeDtypeStruct((HIST, BATCH, EMB_DIM), jnp.float32),
        mesh=mesh,
        compiler_params=pltpu.CompilerParams(use_tc_tiling_on_sc=False),
        scratch_types=[
            pltpu.VMEM((HIST, BATCH_PW), jnp.int32),
            pltpu.VMEM((2, BATCH_PW, EMB_DIM), jnp.float32),
            pltpu.SemaphoreType.DMA,
            pltpu.SemaphoreType.DMA,
        ],
    )
    def k(table_hbm, xt_hbm, out_hbm, idx_v, rows_v, gsem, psem):
        wid = lax.axis_index("s") * NC + lax.axis_index("c")
        bbase = wid * BATCH_PW

        def stage(h, _):
            pltpu.sync_copy(
                xt_hbm.at[h].at[pl.ds(bbase, BATCH_PW)], idx_v.at[h]
            )
            return 0

        lax.fori_loop(0, HIST, stage, 0)

        def step(h, _):
            for t in range(NSTREAM):
                pltpu.async_copy(
                    table_hbm.at[idx_v.at[h].at[pl.ds(t * IDXW, IDXW)]],
                    rows_v.at[0].at[pl.ds(t * IDXW, IDXW)],
                    gsem,
                )
            for t in range(NSTREAM):
                pltpu.make_async_copy(
                    table_hbm.at[idx_v.at[0].at[pl.ds(0, IDXW)]],
                    rows_v.at[0].at[pl.ds(0, IDXW)],
                    gsem,
                ).wait()
            pltpu.async_copy(
                rows_v.at[0],
                out_hbm.at[h].at[pl.ds(bbase, BATCH_PW)],
                psem,
            ).wait()
            return 0

        lax.fori_loop(0, HIST, step, 0)

    return k(table, xt)


def _mlp_body(x_ref, w1_ref, b1_ref, w2_ref, b2_ref, o_ref):
    gt = x_ref[0].T
    h = jnp.dot(w1_ref[...], gt, preferred_element_type=jnp.float32)
    h = jnp.maximum(h + b1_ref[...], 0.0)
    h = jnp.dot(w2_ref[...], h, preferred_element_type=jnp.float32)
    o_ref[0] = jnp.maximum(h + b2_ref[...], 0.0)


def _tc_mlp(gh, w1, b1c, w2, b2c):
    grid = (HIST, BATCH // MLP_B)
    return pl.pallas_call(
        _mlp_body,
        grid=grid,
        in_specs=[
            pl.BlockSpec((1, MLP_B, EMB_DIM), lambda h, c: (h, c, 0)),
            pl.BlockSpec((EMB_DIM, EMB_DIM), lambda h, c: (0, 0)),
            pl.BlockSpec((EMB_DIM, 1), lambda h, c: (0, 0)),
            pl.BlockSpec((EMB_DIM, EMB_DIM), lambda h, c: (0, 0)),
            pl.BlockSpec((EMB_DIM, 1), lambda h, c: (0, 0)),
        ],
        out_specs=pl.BlockSpec((1, EMB_DIM, MLP_B), lambda h, c: (h, 0, c)),
        out_shape=jax.ShapeDtypeStruct((HIST, EMB_DIM, BATCH), jnp.float32),
    )(gh, w1, b1c, w2, b2c)


def kernel(x, emb, W1, b1, W2, b2):
    xt = x.T.astype(jnp.int32)
    gh = _sc_gather(emb, xt)
    out3 = _tc_mlp(
        gh, W1, b1.reshape(EMB_DIM, 1), W2, b2.reshape(EMB_DIM, 1)
    )
    return jnp.transpose(out3, (2, 0, 1))


# speedup vs baseline: 1.5827x; 1.2661x over previous
import functools

import jax
import jax.numpy as jnp
from jax import lax
from jax.experimental import pallas as pl
from jax.experimental.pallas import tpu as pltpu
from jax.experimental.pallas import tpu_sc as plsc

NUM_EMB = 1000000
EMB_DIM = 32
BATCH = 16384
HIST = 50
B = BATCH * HIST

NC, NS = 2, 16
NW = NC * NS
BATCH_PW = BATCH // NW
IDXW = 128
NSTREAM = BATCH_PW // IDXW

MLP_B = 16384


def _sc_gather(table, xt):
    mesh = plsc.VectorSubcoreMesh(core_axis_name="c", subcore_axis_name="s")

    @functools.partial(
        pl.kernel,
        out_type=jax.ShapeDtypeStruct((HIST, BATCH, EMB_DIM), jnp.float32),
        mesh=mesh,
        compiler_params=pltpu.CompilerParams(use_tc_tiling_on_sc=False),
        scratch_types=[
            pltpu.VMEM((HIST, BATCH_PW), jnp.int32),
            pltpu.VMEM((2, BATCH_PW, EMB_DIM), jnp.float32),
            pltpu.SemaphoreType.DMA,
            pltpu.SemaphoreType.DMA,
            pltpu.SemaphoreType.DMA,
            pltpu.SemaphoreType.DMA,
        ],
    )
    def k(table_hbm, xt_hbm, out_hbm, idx_v, rows_v, gsem0, gsem1, psem0, psem1):
        wid = lax.axis_index("s") * NC + lax.axis_index("c")
        bbase = wid * BATCH_PW
        pltpu.sync_copy(xt_hbm.at[:, pl.ds(bbase, BATCH_PW)], idx_v)

        def gathers(h, buf, gsem):
            for t in range(NSTREAM):
                pltpu.async_copy(
                    table_hbm.at[idx_v.at[h].at[pl.ds(t * IDXW, IDXW)]],
                    rows_v.at[buf].at[pl.ds(t * IDXW, IDXW)],
                    gsem,
                )

        def wait_gathers(buf, gsem):
            for t in range(NSTREAM):
                pltpu.make_async_copy(
                    table_hbm.at[idx_v.at[0].at[pl.ds(0, IDXW)]],
                    rows_v.at[buf].at[pl.ds(0, IDXW)],
                    gsem,
                ).wait()

        def put(h, buf, psem):
            pltpu.async_copy(
                rows_v.at[buf], out_hbm.at[h].at[pl.ds(bbase, BATCH_PW)], psem
            )

        def wait_put(buf, psem):
            pltpu.make_async_copy(
                rows_v.at[buf], out_hbm.at[0].at[pl.ds(bbase, BATCH_PW)], psem
            ).wait()

        gathers(0, 0, gsem0)

        def step(i, _):
            h0 = 2 * i
            pl.when(i > 0)(lambda: wait_put(1, psem1))
            gathers(h0 + 1, 1, gsem1)
            wait_gathers(0, gsem0)
            put(h0, 0, psem0)
            wait_put(0, psem0)
            pl.when(i < HIST // 2 - 1)(lambda: gathers(h0 + 2, 0, gsem0))
            wait_gathers(1, gsem1)
            put(h0 + 1, 1, psem1)
            return 0

        lax.fori_loop(0, HIST // 2, step, 0)
        wait_put(1, psem1)

    return k(table, xt)


def _mlp_body(x_ref, w1_ref, b1_ref, w2_ref, b2_ref, o_ref):
    gt = x_ref[0].T
    h = jnp.dot(w1_ref[...], gt, preferred_element_type=jnp.float32)
    h = jnp.maximum(h + b1_ref[...], 0.0)
    h = jnp.dot(w2_ref[...], h, preferred_element_type=jnp.float32)
    o_ref[0] = jnp.maximum(h + b2_ref[...], 0.0)


def _tc_mlp(gh, w1, b1c, w2, b2c):
    grid = (HIST, BATCH // MLP_B)
    return pl.pallas_call(
        _mlp_body,
        grid=grid,
        in_specs=[
            pl.BlockSpec((1, MLP_B, EMB_DIM), lambda h, c: (h, c, 0)),
            pl.BlockSpec((EMB_DIM, EMB_DIM), lambda h, c: (0, 0)),
            pl.BlockSpec((EMB_DIM, 1), lambda h, c: (0, 0)),
            pl.BlockSpec((EMB_DIM, EMB_DIM), lambda h, c: (0, 0)),
            pl.BlockSpec((EMB_DIM, 1), lambda h, c: (0, 0)),
        ],
        out_specs=pl.BlockSpec((1, EMB_DIM, MLP_B), lambda h, c: (h, 0, c)),
        out_shape=jax.ShapeDtypeStruct((HIST, EMB_DIM, BATCH), jnp.float32),
    )(gh, w1, b1c, w2, b2c)


def kernel(x, emb, W1, b1, W2, b2):
    xt = x.T.astype(jnp.int32)
    gh = _sc_gather(emb, xt)
    out3 = _tc_mlp(
        gh, W1, b1.reshape(EMB_DIM, 1), W2, b2.reshape(EMB_DIM, 1)
    )
    return jnp.transpose(out3, (2, 0, 1))
